# baseline (speedup 1.0000x reference)
import jax
import jax.numpy as jnp
from jax import lax
from jax.experimental import pallas as pl
from jax.experimental.pallas import tpu as pltpu
from jax.experimental.pallas import tpu_sc as plsc

N_SRC = 10000
N_DST = 10000
E = 160000
SRC_DIMS = 256
DST_DIMS = 256
HID = 128
HEADS = 4
HH = HEADS * HID

NW = 32
W = 112
NWIN = 48
EPW = W * NWIN
EPAD = NW * EPW
NTILES = 16
NPAD = 10240
STRIPE = NPAD // NTILES
ZW = 80
EB = 896
RB = 1000

_MESH = plsc.VectorSubcoreMesh(core_axis_name="c", subcore_axis_name="s",
                               num_cores=2, num_subcores=16)


def _pack(x):
    lo = lax.bitcast_convert_type(x[:, :HH // 2], jnp.int32)
    hi = lax.bitcast_convert_type(x[:, HH // 2:], jnp.int32)
    plo = (lo + 0x8000) & jnp.int32(-65536)
    phi = lax.shift_right_logical(hi + 0x8000, 16)
    return plo | phi


def _unpack(p):
    lo = lax.bitcast_convert_type(p & jnp.int32(-65536), jnp.float32)
    hi = lax.bitcast_convert_type(lax.shift_left(p, 16), jnp.float32)
    return jnp.concatenate([lo, hi], axis=1)


def _dense_body(sx, dx, ws, bs, wd, bd, wl, bl, wr, br, de_o, xl_o, xr_o):
    se = jax.nn.relu(jnp.dot(sx[...], ws[...], preferred_element_type=jnp.float32) + bs[...])
    de = jax.nn.relu(jnp.dot(dx[...], wd[...], preferred_element_type=jnp.float32) + bd[...])
    de_o[...] = de
    xl = jnp.dot(se, wl[...], preferred_element_type=jnp.float32) + bl[...]
    xr = jnp.dot(de, wr[...], preferred_element_type=jnp.float32) + br[...]
    xl_o[...] = _pack(xl)
    xr_o[...] = _pack(xr)


def _dense(src_x, dst_x, W_src, b_src, W_dst, b_dst, W_l, b_l, W_r, b_r):
    full = lambda shape: pl.BlockSpec(shape, lambda i: (0,) * len(shape))
    return pl.pallas_call(
        _dense_body,
        grid=(N_SRC // RB,),
        in_specs=[
            pl.BlockSpec((RB, SRC_DIMS), lambda i: (i, 0)),
            pl.BlockSpec((RB, DST_DIMS), lambda i: (i, 0)),
            full((SRC_DIMS, HID)), full((1, HID)),
            full((DST_DIMS, HID)), full((1, HID)),
            full((HID, HH)), full((1, HH)),
            full((HID, HH)), full((1, HH)),
        ],
        out_specs=[
            pl.BlockSpec((RB, HID), lambda i: (i, 0)),
            pl.BlockSpec((RB, HH // 2), lambda i: (i, 0)),
            pl.BlockSpec((RB, HH // 2), lambda i: (i, 0)),
        ],
        out_shape=[
            jax.ShapeDtypeStruct((N_DST, HID), jnp.float32),
            jax.ShapeDtypeStruct((N_SRC, HH // 2), jnp.int32),
            jax.ShapeDtypeStruct((N_DST, HH // 2), jnp.int32),
        ],
    )(src_x, dst_x, W_src, b_src.reshape(1, HID), W_dst, b_dst.reshape(1, HID),
      W_l, b_l.reshape(1, HH), W_r, b_r.reshape(1, HH))


def _sc_gather_body(xl_hbm, xr_hbm, si_hbm, di_hbm, xls_hbm, xrs_hbm,
                    ia0, ia1, a0, a1, b0, b1,
                    gs0, gs1, gs2, gs3, ws0, ws1, ws2, ws3):
    c = lax.axis_index("c")
    s = lax.axis_index("s")
    ebase = pl.multiple_of((s * 2 + c) * EPW, 8)
    pltpu.sync_copy(si_hbm.at[pl.ds(ebase, EPW)], ia0)
    pltpu.sync_copy(di_hbm.at[pl.ds(ebase, EPW)], ia1)

    def dsts(w):
        base = pl.multiple_of(ebase + w * W, 8)
        return xls_hbm.at[pl.ds(base, W), :], xrs_hbm.at[pl.ds(base, W), :]

    def gather(w, bx, br, sx, sr):
        pltpu.async_copy(xl_hbm.at[ia0.at[pl.ds(w * W, W)]], bx, sx)
        pltpu.async_copy(xr_hbm.at[ia1.at[pl.ds(w * W, W)]], br, sr)

    gather(0, a0, a1, gs0, gs1)

    def pair(k, carry):
        w0 = k * 2
        w1 = w0 + 1
        d0 = dsts(w0)
        d1 = dsts(w1)

        @pl.when(k > 0)
        def _():
            pd = dsts(w0 - 1)
            pltpu.make_async_copy(b0, pd[0], ws2).wait()
            pltpu.make_async_copy(b1, pd[1], ws3).wait()

        gather(w1, b0, b1, gs2, gs3)
        pltpu.make_async_copy(xl_hbm.at[ia0.at[pl.ds(w0 * W, W)]], a0, gs0).wait()
        pltpu.make_async_copy(xr_hbm.at[ia1.at[pl.ds(w0 * W, W)]], a1, gs1).wait()
        pltpu.async_copy(a0, d0[0], ws0)
        pltpu.async_copy(a1, d0[1], ws1)
        pltpu.make_async_copy(a0, d0[0], ws0).wait()
        pltpu.make_async_copy(a1, d0[1], ws1).wait()

        @pl.when(k < NWIN // 2 - 1)
        def _():
            gather(w0 + 2, a0, a1, gs0, gs1)

        pltpu.make_async_copy(xl_hbm.at[ia0.at[pl.ds(w1 * W, W)]], b0, gs2).wait()
        pltpu.make_async_copy(xr_hbm.at[ia1.at[pl.ds(w1 * W, W)]], b1, gs3).wait()
        pltpu.async_copy(b0, d1[0], ws2)
        pltpu.async_copy(b1, d1[1], ws3)
        return carry

    lax.fori_loop(0, NWIN // 2, pair, 0)
    last = dsts(NWIN - 1)
    pltpu.make_async_copy(b0, last[0], ws2).wait()
    pltpu.make_async_copy(b1, last[1], ws3).wait()


_sc_gather = pl.kernel(
    _sc_gather_body,
    out_type=[
        jax.ShapeDtypeStruct((EPAD, HH // 2), jnp.int32),
        jax.ShapeDtypeStruct((EPAD, HH // 2), jnp.int32),
    ],
    mesh=_MESH,
    scratch_types=[
        pltpu.VMEM((EPW,), jnp.int32),
        pltpu.VMEM((EPW,), jnp.int32),
        pltpu.VMEM((W, HH // 2), jnp.int32),
        pltpu.VMEM((W, HH // 2), jnp.int32),
        pltpu.VMEM((W, HH // 2), jnp.int32),
        pltpu.VMEM((W, HH // 2), jnp.int32),
    ] + [pltpu.SemaphoreType.DMA] * 8,
)


def _alpha_body(xls, xrs, att, ex_o):
    pid = pl.program_id(0)
    xlv = _unpack(xls[...])
    xrv = _unpack(xrs[...])
    cols = []
    for h in range(HEADS):
        x = xlv[:, h * HID:(h + 1) * HID] + xrv[:, h * HID:(h + 1) * HID]
        x = jnp.where(x > 0, x, 0.2 * x)
        a = jnp.sum(x * att[h, :][None, :], axis=1, keepdims=True)
        cols.append(jnp.exp(a))
    ex = jnp.concatenate(cols + [jnp.zeros((EB, HID - HEADS), jnp.float32)], axis=1)
    eid = pid * EB + lax.broadcasted_iota(jnp.int32, (EB, 1), 0)
    ex_o[...] = jnp.where(eid < E, ex, 0.0)


def _alpha(xls, xrs, att):
    return pl.pallas_call(
        _alpha_body,
        grid=(EPAD // EB,),
        in_specs=[
            pl.BlockSpec((EB, HH // 2), lambda i: (i, 0)),
            pl.BlockSpec((EB, HH // 2), lambda i: (i, 0)),
            pl.BlockSpec((HEADS, HID), lambda i: (0, 0)),
        ],
        out_specs=pl.BlockSpec((EB, HID), lambda i: (i, 0)),
        out_shape=jax.ShapeDtypeStruct((EPAD, HID), jnp.float32),
    )(xls, xrs, att)


EPT = EPAD // NTILES
NWIN_T = EPT // W


def _sc_denom_body(exa_hbm, di_hbm, dd_hbm, den_sp,
                   ex0, ex1, ddbuf, ix0, ix1, ssem0, ssem1):
    c = lax.axis_index("c")
    s = lax.axis_index("s")
    wid = s * 2 + c
    z = jnp.zeros((16,), jnp.float32)

    def zrow(i, carry):
        for j in range(HID // 16):
            ex0[i, pl.ds(j * 16, 16)] = z
        return carry

    lax.fori_loop(0, ZW, zrow, 0)
    for k in range(STRIPE // ZW):
        pltpu.sync_copy(ex0.at[pl.ds(0, ZW)], den_sp.at[pl.ds(s * STRIPE + k * ZW, ZW)])
    plsc.subcore_barrier()

    def win(w2, carry):
        for j, (exb, ixb, ssem) in ((0, (ex0, ix0, ssem0)), (1, (ex1, ix1, ssem1))):
            base = pl.multiple_of(s * EPT + (w2 * 2 + j) * W, 8)

            @pl.when(w2 > 0)
            def _():
                pltpu.make_async_copy(exb, den_sp.at[ixb], ssem).wait()

            pltpu.sync_copy(exa_hbm.at[pl.ds(base, W), :], exb)
            pltpu.sync_copy(di_hbm.at[pl.ds(base, W)], ixb)
            pltpu.async_copy(exb, den_sp.at[ixb], ssem, add=True)
        return carry

    lax.fori_loop(0, NWIN_T // 2, win, 0)
    pltpu.make_async_copy(ex0, den_sp.at[ix0], ssem0).wait()
    pltpu.make_async_copy(ex1, den_sp.at[ix1], ssem1).wait()
    plsc.subcore_barrier()

    def win2(w, carry):
        base = pl.multiple_of(wid * EPW + w * W, 8)
        pltpu.sync_copy(di_hbm.at[pl.ds(base, W)], ix0)
        pltpu.sync_copy(den_sp.at[ix0], ddbuf)
        pltpu.sync_copy(ddbuf, dd_hbm.at[pl.ds(base, W), :])
        return carry

    lax.fori_loop(0, NWIN, win2, 0)


_sc_denom = pl.kernel(
    _sc_denom_body,
    out_type=jax.ShapeDtypeStruct((EPAD, HID), jnp.float32),
    mesh=_MESH,
    scratch_types=[
        pltpu.VMEM_SHARED((NPAD, HID), jnp.float32),
        pltpu.VMEM((W, HID), jnp.float32),
        pltpu.VMEM((W, HID), jnp.float32),
        pltpu.VMEM((W, HID), jnp.float32),
        pltpu.VMEM((W,), jnp.int32),
        pltpu.VMEM((W,), jnp.int32),
        pltpu.SemaphoreType.DMA,
        pltpu.SemaphoreType.DMA,
    ],
)


def _msg_body(xls, exa, dd, msg_o):
    xlv = _unpack(xls[...])
    pieces = []
    for h in range(HEADS):
        coef = exa[:, h:h + 1] / (dd[:, h:h + 1] + 1e-16)
        pieces.append(xlv[:, h * HID:(h + 1) * HID] * coef)
    msg_o[...] = jnp.concatenate(pieces, axis=1)


def _msg(xls, exa, dd):
    return pl.pallas_call(
        _msg_body,
        grid=(EPAD // EB,),
        in_specs=[
            pl.BlockSpec((EB, HH // 2), lambda i: (i, 0)),
            pl.BlockSpec((EB, HID), lambda i: (i, 0)),
            pl.BlockSpec((EB, HID), lambda i: (i, 0)),
        ],
        out_specs=pl.BlockSpec((EB, HH), lambda i: (i, 0)),
        out_shape=jax.ShapeDtypeStruct((EPAD, HH), jnp.float32),
    )(xls, exa, dd)


def _sc_scatter_body(msg_hbm, di_hbm, op_hbm, out_sp,
                     zbuf, mb0, mb1, ix0, ix1, ssem0, ssem1):
    c = lax.axis_index("c")
    s = lax.axis_index("s")
    wid = s * 2 + c
    z = jnp.zeros((16,), jnp.float32)

    def zrow(i, carry):
        for j in range(HID // 16):
            zbuf[i, pl.ds(j * 16, 16)] = z
        return carry

    lax.fori_loop(0, ZW, zrow, 0)

    for h in range(HEADS):
        for k in range(STRIPE // ZW):
            pltpu.sync_copy(zbuf, out_sp.at[pl.ds(s * STRIPE + k * ZW, ZW)])
        plsc.subcore_barrier()

        def win(w2, carry):
            for j, (mb, ixb, ssem) in ((0, (mb0, ix0, ssem0)), (1, (mb1, ix1, ssem1))):
                base = pl.multiple_of(wid * EPW + (w2 * 2 + j) * W, 8)

                @pl.when(w2 > 0)
                def _():
                    pltpu.make_async_copy(mb, out_sp.at[ixb], ssem).wait()

                pltpu.sync_copy(msg_hbm.at[pl.ds(base, W), pl.ds(h * HID, HID)], mb)
                pltpu.sync_copy(di_hbm.at[pl.ds(base, W)], ixb)
                pltpu.async_copy(mb, out_sp.at[ixb], ssem, add=True)
            return carry

        lax.fori_loop(0, NWIN // 2, win, 0)
        pltpu.make_async_copy(mb0, out_sp.at[ix0], ssem0).wait()
        pltpu.make_async_copy(mb1, out_sp.at[ix1], ssem1).wait()
        plsc.subcore_barrier()
        pltpu.sync_copy(out_sp.at[pl.ds(s * STRIPE, STRIPE)],
                        op_hbm.at[c * HEADS + h, pl.ds(s * STRIPE, STRIPE), :])


_sc_scatter = pl.kernel(
    _sc_scatter_body,
    out_type=jax.ShapeDtypeStruct((2 * HEADS, NPAD, HID), jnp.float32),
    mesh=_MESH,
    scratch_types=[
        pltpu.VMEM_SHARED((NPAD, HID), jnp.float32),
        pltpu.VMEM((ZW, HID), jnp.float32),
        pltpu.VMEM((W, HID), jnp.float32),
        pltpu.VMEM((W, HID), jnp.float32),
        pltpu.VMEM((W,), jnp.int32),
        pltpu.VMEM((W,), jnp.int32),
        pltpu.SemaphoreType.DMA,
        pltpu.SemaphoreType.DMA,
    ],
)


def _final_body(de, op, bo, out_o):
    pieces = [de[...]]
    for h in range(HEADS):
        v = op[h] + op[HEADS + h] + bo[:, h * HID:(h + 1) * HID]
        pieces.append(jax.nn.relu(v))
    out_o[...] = jnp.concatenate(pieces, axis=1)


def _final(de, op, bias_out):
    return pl.pallas_call(
        _final_body,
        grid=(N_DST // RB,),
        in_specs=[
            pl.BlockSpec((RB, HID), lambda i: (i, 0)),
            pl.BlockSpec((2 * HEADS, RB, HID), lambda i: (0, i, 0)),
            pl.BlockSpec((1, HH), lambda i: (0, 0)),
        ],
        out_specs=pl.BlockSpec((RB, HID + HH), lambda i: (i, 0)),
        out_shape=jax.ShapeDtypeStruct((N_DST, HID + HH), jnp.float32),
    )(de, op, bias_out.reshape(1, HH))


def kernel(src_x, dst_x, edge_index, W_src, b_src, W_dst, b_dst,
           W_l, b_l, W_r, b_r, att, bias_out):
    ei = jnp.pad(edge_index, ((0, 0), (0, EPAD - E)))
    si = ei[0]
    di = ei[1]
    de, xl, xr = _dense(src_x, dst_x, W_src, b_src, W_dst, b_dst, W_l, b_l, W_r, b_r)
    xls, xrs = _sc_gather(xl, xr, si, di)
    exa = _alpha(xls, xrs, att)
    dd = _sc_denom(exa, di)
    msg = _msg(xls, exa, dd)
    op = _sc_scatter(msg, di)
    return _final(de, op, bias_out)


# speedup vs baseline: 1.0618x; 1.0618x over previous
import jax
import jax.numpy as jnp
from jax import lax
from jax.experimental import pallas as pl
from jax.experimental.pallas import tpu as pltpu
from jax.experimental.pallas import tpu_sc as plsc

N_SRC = 10000
N_DST = 10000
E = 160000
SRC_DIMS = 256
DST_DIMS = 256
HID = 128
HEADS = 4
HH = HEADS * HID

NW = 32
W = 120
NWIN = 44
EPW = W * NWIN
EPAD = NW * EPW
NTILES = 16
NPAD = 10240
STRIPE = NPAD // NTILES
ZW = 80
EB = 960
RB = 1000

_MESH = plsc.VectorSubcoreMesh(core_axis_name="c", subcore_axis_name="s",
                               num_cores=2, num_subcores=16)


def _pack(x):
    lo = lax.bitcast_convert_type(x[:, :HH // 2], jnp.int32)
    hi = lax.bitcast_convert_type(x[:, HH // 2:], jnp.int32)
    plo = (lo + 0x8000) & jnp.int32(-65536)
    phi = lax.shift_right_logical(hi + 0x8000, 16)
    return plo | phi


def _unpack(p):
    lo = lax.bitcast_convert_type(p & jnp.int32(-65536), jnp.float32)
    hi = lax.bitcast_convert_type(lax.shift_left(p, 16), jnp.float32)
    return jnp.concatenate([lo, hi], axis=1)


def _dense_body(sx, dx, ws, bs, wd, bd, wl, bl, wr, br, de_o, xl_o, xr_o):
    se = jax.nn.relu(jnp.dot(sx[...], ws[...], preferred_element_type=jnp.float32) + bs[...])
    de = jax.nn.relu(jnp.dot(dx[...], wd[...], preferred_element_type=jnp.float32) + bd[...])
    de_o[...] = de
    xl = jnp.dot(se, wl[...], preferred_element_type=jnp.float32) + bl[...]
    xr = jnp.dot(de, wr[...], preferred_element_type=jnp.float32) + br[...]
    xl_o[...] = _pack(xl)
    xr_o[...] = _pack(xr)


def _dense(src_x, dst_x, W_src, b_src, W_dst, b_dst, W_l, b_l, W_r, b_r):
    full = lambda shape: pl.BlockSpec(shape, lambda i: (0,) * len(shape))
    return pl.pallas_call(
        _dense_body,
        grid=(N_SRC // RB,),
        in_specs=[
            pl.BlockSpec((RB, SRC_DIMS), lambda i: (i, 0)),
            pl.BlockSpec((RB, DST_DIMS), lambda i: (i, 0)),
            full((SRC_DIMS, HID)), full((1, HID)),
            full((DST_DIMS, HID)), full((1, HID)),
            full((HID, HH)), full((1, HH)),
            full((HID, HH)), full((1, HH)),
        ],
        out_specs=[
            pl.BlockSpec((RB, HID), lambda i: (i, 0)),
            pl.BlockSpec((RB, HH // 2), lambda i: (i, 0)),
            pl.BlockSpec((RB, HH // 2), lambda i: (i, 0)),
        ],
        out_shape=[
            jax.ShapeDtypeStruct((N_DST, HID), jnp.float32),
            jax.ShapeDtypeStruct((N_SRC, HH // 2), jnp.int32),
            jax.ShapeDtypeStruct((N_DST, HH // 2), jnp.int32),
        ],
    )(src_x, dst_x, W_src, b_src.reshape(1, HID), W_dst, b_dst.reshape(1, HID),
      W_l, b_l.reshape(1, HH), W_r, b_r.reshape(1, HH))


def _sc_gather_body(xl_hbm, xr_hbm, si_hbm, di_hbm, xls_hbm, xrs_hbm,
                    idx0, idx1, buf0, buf1, gsem0, gsem1, wsem0, wsem1):
    c = lax.axis_index("c")
    s = lax.axis_index("s")
    wid = s * 2 + c

    def win(w, carry):
        base = pl.multiple_of(wid * EPW + w * W, 8)
        dst0 = xls_hbm.at[pl.ds(base, W), :]
        dst1 = xrs_hbm.at[pl.ds(base, W), :]

        @pl.when(w > 0)
        def _():
            pltpu.make_async_copy(buf0, dst0, wsem0).wait()
            pltpu.make_async_copy(buf1, dst1, wsem1).wait()

        pltpu.sync_copy(si_hbm.at[pl.ds(base, W)], idx0)
        g0 = pltpu.async_copy(xl_hbm.at[idx0], buf0, gsem0)
        pltpu.sync_copy(di_hbm.at[pl.ds(base, W)], idx1)
        g1 = pltpu.async_copy(xr_hbm.at[idx1], buf1, gsem1)
        g0.wait()
        pltpu.async_copy(buf0, dst0, wsem0)
        g1.wait()
        pltpu.async_copy(buf1, dst1, wsem1)
        return carry

    lax.fori_loop(0, NWIN, win, 0)
    last = pl.multiple_of(wid * EPW + (NWIN - 1) * W, 8)
    pltpu.make_async_copy(buf0, xls_hbm.at[pl.ds(last, W), :], wsem0).wait()
    pltpu.make_async_copy(buf1, xrs_hbm.at[pl.ds(last, W), :], wsem1).wait()


_sc_gather = pl.kernel(
    _sc_gather_body,
    out_type=[
        jax.ShapeDtypeStruct((EPAD, HH // 2), jnp.int32),
        jax.ShapeDtypeStruct((EPAD, HH // 2), jnp.int32),
    ],
    mesh=_MESH,
    scratch_types=[
        pltpu.VMEM((W,), jnp.int32),
        pltpu.VMEM((W,), jnp.int32),
        pltpu.VMEM((W, HH // 2), jnp.int32),
        pltpu.VMEM((W, HH // 2), jnp.int32),
        pltpu.SemaphoreType.DMA,
        pltpu.SemaphoreType.DMA,
        pltpu.SemaphoreType.DMA,
        pltpu.SemaphoreType.DMA,
    ],
)


def _alpha_body(xls, xrs, att, ex_o):
    pid = pl.program_id(0)
    xlv = _unpack(xls[...])
    xrv = _unpack(xrs[...])
    cols = []
    for h in range(HEADS):
        x = xlv[:, h * HID:(h + 1) * HID] + xrv[:, h * HID:(h + 1) * HID]
        x = jnp.where(x > 0, x, 0.2 * x)
        a = jnp.sum(x * att[h, :][None, :], axis=1, keepdims=True)
        cols.append(jnp.exp(a))
    ex = jnp.concatenate(cols + [jnp.zeros((EB, HID - HEADS), jnp.float32)], axis=1)
    eid = pid * EB + lax.broadcasted_iota(jnp.int32, (EB, 1), 0)
    ex_o[...] = jnp.where(eid < E, ex, 0.0)


def _alpha(xls, xrs, att):
    return pl.pallas_call(
        _alpha_body,
        grid=(EPAD // EB,),
        in_specs=[
            pl.BlockSpec((EB, HH // 2), lambda i: (i, 0)),
            pl.BlockSpec((EB, HH // 2), lambda i: (i, 0)),
            pl.BlockSpec((HEADS, HID), lambda i: (0, 0)),
        ],
        out_specs=pl.BlockSpec((EB, HID), lambda i: (i, 0)),
        out_shape=jax.ShapeDtypeStruct((EPAD, HID), jnp.float32),
    )(xls, xrs, att)


EPT = EPAD // NTILES
NWIN_T = EPT // W


def _sc_denom_body(exa_hbm, di_hbm, dd_hbm, den_sp,
                   ex0, ex1, ddbuf, ix0, ix1, ssem0, ssem1):
    c = lax.axis_index("c")
    s = lax.axis_index("s")
    wid = s * 2 + c
    z = jnp.zeros((16,), jnp.float32)

    def zrow(i, carry):
        for j in range(HID // 16):
            ex0[i, pl.ds(j * 16, 16)] = z
        return carry

    lax.fori_loop(0, ZW, zrow, 0)
    for k in range(STRIPE // ZW):
        pltpu.sync_copy(ex0.at[pl.ds(0, ZW)], den_sp.at[pl.ds(s * STRIPE + k * ZW, ZW)])
    plsc.subcore_barrier()

    def win(w2, carry):
        for j, (exb, ixb, ssem) in ((0, (ex0, ix0, ssem0)), (1, (ex1, ix1, ssem1))):
            base = pl.multiple_of(s * EPT + (w2 * 2 + j) * W, 8)

            @pl.when(w2 > 0)
            def _():
                pltpu.make_async_copy(exb, den_sp.at[ixb], ssem).wait()

            pltpu.sync_copy(exa_hbm.at[pl.ds(base, W), :], exb)
            pltpu.sync_copy(di_hbm.at[pl.ds(base, W)], ixb)
            pltpu.async_copy(exb, den_sp.at[ixb], ssem, add=True)
        return carry

    lax.fori_loop(0, NWIN_T // 2, win, 0)
    pltpu.make_async_copy(ex0, den_sp.at[ix0], ssem0).wait()
    pltpu.make_async_copy(ex1, den_sp.at[ix1], ssem1).wait()
    plsc.subcore_barrier()

    def win2(w, carry):
        base = pl.multiple_of(wid * EPW + w * W, 8)
        pltpu.sync_copy(di_hbm.at[pl.ds(base, W)], ix0)
        pltpu.sync_copy(den_sp.at[ix0], ddbuf)
        pltpu.sync_copy(ddbuf, dd_hbm.at[pl.ds(base, W), :])
        return carry

    lax.fori_loop(0, NWIN, win2, 0)


_sc_denom = pl.kernel(
    _sc_denom_body,
    out_type=jax.ShapeDtypeStruct((EPAD, HID), jnp.float32),
    mesh=_MESH,
    scratch_types=[
        pltpu.VMEM_SHARED((NPAD, HID), jnp.float32),
        pltpu.VMEM((W, HID), jnp.float32),
        pltpu.VMEM((W, HID), jnp.float32),
        pltpu.VMEM((W, HID), jnp.float32),
        pltpu.VMEM((W,), jnp.int32),
        pltpu.VMEM((W,), jnp.int32),
        pltpu.SemaphoreType.DMA,
        pltpu.SemaphoreType.DMA,
    ],
)


def _msg_body(xls, exa, dd, msg_o):
    xlv = _unpack(xls[...])
    pieces = []
    for h in range(HEADS):
        coef = exa[:, h:h + 1] / (dd[:, h:h + 1] + 1e-16)
        pieces.append(xlv[:, h * HID:(h + 1) * HID] * coef)
    msg_o[...] = jnp.concatenate(pieces, axis=1)


def _msg(xls, exa, dd):
    return pl.pallas_call(
        _msg_body,
        grid=(EPAD // EB,),
        in_specs=[
            pl.BlockSpec((EB, HH // 2), lambda i: (i, 0)),
            pl.BlockSpec((EB, HID), lambda i: (i, 0)),
            pl.BlockSpec((EB, HID), lambda i: (i, 0)),
        ],
        out_specs=pl.BlockSpec((EB, HH), lambda i: (i, 0)),
        out_shape=jax.ShapeDtypeStruct((EPAD, HH), jnp.float32),
    )(xls, exa, dd)


def _sc_scatter_body(msg_hbm, di_hbm, op_hbm, out_sp,
                     zbuf, mb0, mb1, ix0, ix1, ssem0, ssem1):
    c = lax.axis_index("c")
    s = lax.axis_index("s")
    wid = s * 2 + c
    z = jnp.zeros((16,), jnp.float32)

    def zrow(i, carry):
        for j in range(HID // 16):
            zbuf[i, pl.ds(j * 16, 16)] = z
        return carry

    lax.fori_loop(0, ZW, zrow, 0)

    for h in range(HEADS):
        for k in range(STRIPE // ZW):
            pltpu.sync_copy(zbuf, out_sp.at[pl.ds(s * STRIPE + k * ZW, ZW)])
        plsc.subcore_barrier()

        def win(w2, carry):
            for j, (mb, ixb, ssem) in ((0, (mb0, ix0, ssem0)), (1, (mb1, ix1, ssem1))):
                base = pl.multiple_of(wid * EPW + (w2 * 2 + j) * W, 8)

                @pl.when(w2 > 0)
                def _():
                    pltpu.make_async_copy(mb, out_sp.at[ixb], ssem).wait()

                pltpu.sync_copy(msg_hbm.at[pl.ds(base, W), pl.ds(h * HID, HID)], mb)
                pltpu.sync_copy(di_hbm.at[pl.ds(base, W)], ixb)
                pltpu.async_copy(mb, out_sp.at[ixb], ssem, add=True)
            return carry

        lax.fori_loop(0, NWIN // 2, win, 0)
        pltpu.make_async_copy(mb0, out_sp.at[ix0], ssem0).wait()
        pltpu.make_async_copy(mb1, out_sp.at[ix1], ssem1).wait()
        plsc.subcore_barrier()
        pltpu.sync_copy(out_sp.at[pl.ds(s * STRIPE, STRIPE)],
                        op_hbm.at[c * HEADS + h, pl.ds(s * STRIPE, STRIPE), :])


_sc_scatter = pl.kernel(
    _sc_scatter_body,
    out_type=jax.ShapeDtypeStruct((2 * HEADS, NPAD, HID), jnp.float32),
    mesh=_MESH,
    scratch_types=[
        pltpu.VMEM_SHARED((NPAD, HID), jnp.float32),
        pltpu.VMEM((ZW, HID), jnp.float32),
        pltpu.VMEM((W, HID), jnp.float32),
        pltpu.VMEM((W, HID), jnp.float32),
        pltpu.VMEM((W,), jnp.int32),
        pltpu.VMEM((W,), jnp.int32),
        pltpu.SemaphoreType.DMA,
        pltpu.SemaphoreType.DMA,
    ],
)


def _final_body(de, op, bo, out_o):
    pieces = [de[...]]
    for h in range(HEADS):
        v = op[h] + op[HEADS + h] + bo[:, h * HID:(h + 1) * HID]
        pieces.append(jax.nn.relu(v))
    out_o[...] = jnp.concatenate(pieces, axis=1)


def _final(de, op, bias_out):
    return pl.pallas_call(
        _final_body,
        grid=(N_DST // RB,),
        in_specs=[
            pl.BlockSpec((RB, HID), lambda i: (i, 0)),
            pl.BlockSpec((2 * HEADS, RB, HID), lambda i: (0, i, 0)),
            pl.BlockSpec((1, HH), lambda i: (0, 0)),
        ],
        out_specs=pl.BlockSpec((RB, HID + HH), lambda i: (i, 0)),
        out_shape=jax.ShapeDtypeStruct((N_DST, HID + HH), jnp.float32),
    )(de, op, bias_out.reshape(1, HH))


def kernel(src_x, dst_x, edge_index, W_src, b_src, W_dst, b_dst,
           W_l, b_l, W_r, b_r, att, bias_out):
    ei = jnp.pad(edge_index, ((0, 0), (0, EPAD - E)))
    si = ei[0]
    di = ei[1]
    de, xl, xr = _dense(src_x, dst_x, W_src, b_src, W_dst, b_dst, W_l, b_l, W_r, b_r)
    xls, xrs = _sc_gather(xl, xr, si, di)
    exa = _alpha(xls, xrs, att)
    dd = _sc_denom(exa, di)
    msg = _msg(xls, exa, dd)
    op = _sc_scatter(msg, di)
    return _final(de, op, bias_out)


# speedup vs baseline: 1.2577x; 1.1845x over previous
import jax
import jax.numpy as jnp
from jax import lax
from jax.experimental import pallas as pl
from jax.experimental.pallas import tpu as pltpu
from jax.experimental.pallas import tpu_sc as plsc

N_SRC = 10000
N_DST = 10000
E = 160000
SRC_DIMS = 256
DST_DIMS = 256
HID = 128
HEADS = 4
HH = HEADS * HID

NW = 32
W = 120
NWIN = 44
EPW = W * NWIN
EPAD = NW * EPW
NTILES = 16
NPAD = 10240
STRIPE = NPAD // NTILES
ZW = 80
EB = 960
RB = 1000

_MESH = plsc.VectorSubcoreMesh(core_axis_name="c", subcore_axis_name="s",
                               num_cores=2, num_subcores=16)


def _pack(x):
    lo = lax.bitcast_convert_type(x[:, :HH // 2], jnp.int32)
    hi = lax.bitcast_convert_type(x[:, HH // 2:], jnp.int32)
    plo = (lo + 0x8000) & jnp.int32(-65536)
    phi = lax.shift_right_logical(hi + 0x8000, 16)
    return plo | phi


def _unpack(p):
    lo = lax.bitcast_convert_type(p & jnp.int32(-65536), jnp.float32)
    hi = lax.bitcast_convert_type(lax.shift_left(p, 16), jnp.float32)
    return jnp.concatenate([lo, hi], axis=1)


def _dense_body(sx, dx, ws, bs, wd, bd, wl, bl, wr, br, de_o, xl_o, xr_o):
    se = jax.nn.relu(jnp.dot(sx[...], ws[...], preferred_element_type=jnp.float32) + bs[...])
    de = jax.nn.relu(jnp.dot(dx[...], wd[...], preferred_element_type=jnp.float32) + bd[...])
    de_o[...] = de
    xl = jnp.dot(se, wl[...], preferred_element_type=jnp.float32) + bl[...]
    xr = jnp.dot(de, wr[...], preferred_element_type=jnp.float32) + br[...]
    xl_o[...] = _pack(xl)
    xr_o[...] = _pack(xr)


def _dense(src_x, dst_x, W_src, b_src, W_dst, b_dst, W_l, b_l, W_r, b_r):
    full = lambda shape: pl.BlockSpec(shape, lambda i: (0,) * len(shape))
    return pl.pallas_call(
        _dense_body,
        grid=(N_SRC // RB,),
        in_specs=[
            pl.BlockSpec((RB, SRC_DIMS), lambda i: (i, 0)),
            pl.BlockSpec((RB, DST_DIMS), lambda i: (i, 0)),
            full((SRC_DIMS, HID)), full((1, HID)),
            full((DST_DIMS, HID)), full((1, HID)),
            full((HID, HH)), full((1, HH)),
            full((HID, HH)), full((1, HH)),
        ],
        out_specs=[
            pl.BlockSpec((RB, HID), lambda i: (i, 0)),
            pl.BlockSpec((RB, HH // 2), lambda i: (i, 0)),
            pl.BlockSpec((RB, HH // 2), lambda i: (i, 0)),
        ],
        out_shape=[
            jax.ShapeDtypeStruct((N_DST, HID), jnp.float32),
            jax.ShapeDtypeStruct((N_SRC, HH // 2), jnp.int32),
            jax.ShapeDtypeStruct((N_DST, HH // 2), jnp.int32),
        ],
    )(src_x, dst_x, W_src, b_src.reshape(1, HID), W_dst, b_dst.reshape(1, HID),
      W_l, b_l.reshape(1, HH), W_r, b_r.reshape(1, HH))


def _sc_gather_body(xl_hbm, xr_hbm, si_hbm, di_hbm, xls_hbm, xrs_hbm,
                    idx0, idx1, buf0, buf1, gsem0, gsem1, wsem0, wsem1):
    c = lax.axis_index("c")
    s = lax.axis_index("s")
    wid = s * 2 + c

    def win(w, carry):
        base = pl.multiple_of(wid * EPW + w * W, 8)
        dst0 = xls_hbm.at[pl.ds(base, W), :]
        dst1 = xrs_hbm.at[pl.ds(base, W), :]

        @pl.when(w > 0)
        def _():
            pltpu.make_async_copy(buf0, dst0, wsem0).wait()
            pltpu.make_async_copy(buf1, dst1, wsem1).wait()

        pltpu.sync_copy(si_hbm.at[pl.ds(base, W)], idx0)
        g0 = pltpu.async_copy(xl_hbm.at[idx0], buf0, gsem0)
        pltpu.sync_copy(di_hbm.at[pl.ds(base, W)], idx1)
        g1 = pltpu.async_copy(xr_hbm.at[idx1], buf1, gsem1)
        g0.wait()
        pltpu.async_copy(buf0, dst0, wsem0)
        g1.wait()
        pltpu.async_copy(buf1, dst1, wsem1)
        return carry

    lax.fori_loop(0, NWIN, win, 0)
    last = pl.multiple_of(wid * EPW + (NWIN - 1) * W, 8)
    pltpu.make_async_copy(buf0, xls_hbm.at[pl.ds(last, W), :], wsem0).wait()
    pltpu.make_async_copy(buf1, xrs_hbm.at[pl.ds(last, W), :], wsem1).wait()


_sc_gather = pl.kernel(
    _sc_gather_body,
    out_type=[
        jax.ShapeDtypeStruct((EPAD, HH // 2), jnp.int32),
        jax.ShapeDtypeStruct((EPAD, HH // 2), jnp.int32),
    ],
    mesh=_MESH,
    scratch_types=[
        pltpu.VMEM((W,), jnp.int32),
        pltpu.VMEM((W,), jnp.int32),
        pltpu.VMEM((W, HH // 2), jnp.int32),
        pltpu.VMEM((W, HH // 2), jnp.int32),
        pltpu.SemaphoreType.DMA,
        pltpu.SemaphoreType.DMA,
        pltpu.SemaphoreType.DMA,
        pltpu.SemaphoreType.DMA,
    ],
)


def _alpha_body(xls, xrs, att, ex_o, msg_o):
    pid = pl.program_id(0)
    xlv = _unpack(xls[...])
    xrv = _unpack(xrs[...])
    eid = pid * EB + lax.broadcasted_iota(jnp.int32, (EB, 1), 0)
    live = eid < E
    cols = []
    pieces = []
    for h in range(HEADS):
        xlh = xlv[:, h * HID:(h + 1) * HID]
        x = xlh + xrv[:, h * HID:(h + 1) * HID]
        x = jnp.where(x > 0, x, 0.2 * x)
        a = jnp.sum(x * att[h, :][None, :], axis=1, keepdims=True)
        ex = jnp.where(live, jnp.exp(a), 0.0)
        cols.append(ex)
        pieces.append(xlh * ex)
    ex_o[...] = jnp.concatenate(cols + [jnp.zeros((EB, HID - HEADS), jnp.float32)], axis=1)
    msg_o[...] = jnp.concatenate(pieces, axis=1)


def _alpha(xls, xrs, att):
    return pl.pallas_call(
        _alpha_body,
        grid=(EPAD // EB,),
        in_specs=[
            pl.BlockSpec((EB, HH // 2), lambda i: (i, 0)),
            pl.BlockSpec((EB, HH // 2), lambda i: (i, 0)),
            pl.BlockSpec((HEADS, HID), lambda i: (0, 0)),
        ],
        out_specs=[
            pl.BlockSpec((EB, HID), lambda i: (i, 0)),
            pl.BlockSpec((EB, HH), lambda i: (i, 0)),
        ],
        out_shape=[
            jax.ShapeDtypeStruct((EPAD, HID), jnp.float32),
            jax.ShapeDtypeStruct((EPAD, HH), jnp.float32),
        ],
    )(xls, xrs, att)


def _sc_denom_body(exa_hbm, di_hbm, den_hbm, den_sp,
                   ex0, ex1, ix0, ix1, ssem0, ssem1):
    c = lax.axis_index("c")
    s = lax.axis_index("s")
    wid2 = c * NTILES + s
    z = jnp.zeros((16,), jnp.float32)

    def zrow(i, carry):
        for j in range(HID // 16):
            ex0[i, pl.ds(j * 16, 16)] = z
        return carry

    lax.fori_loop(0, ZW, zrow, 0)
    for k in range(STRIPE // ZW):
        pltpu.sync_copy(ex0.at[pl.ds(0, ZW)], den_sp.at[pl.ds(s * STRIPE + k * ZW, ZW)])
    plsc.subcore_barrier()

    def win(w2, carry):
        for j, (exb, ixb, ssem) in ((0, (ex0, ix0, ssem0)), (1, (ex1, ix1, ssem1))):
            base = pl.multiple_of(wid2 * EPW + (w2 * 2 + j) * W, 8)

            @pl.when(w2 > 0)
            def _():
                pltpu.make_async_copy(exb, den_sp.at[ixb], ssem).wait()

            pltpu.sync_copy(exa_hbm.at[pl.ds(base, W), :], exb)
            pltpu.sync_copy(di_hbm.at[pl.ds(base, W)], ixb)
            pltpu.async_copy(exb, den_sp.at[ixb], ssem, add=True)
        return carry

    lax.fori_loop(0, NWIN // 2, win, 0)
    pltpu.make_async_copy(ex0, den_sp.at[ix0], ssem0).wait()
    pltpu.make_async_copy(ex1, den_sp.at[ix1], ssem1).wait()
    plsc.subcore_barrier()
    pltpu.sync_copy(den_sp.at[pl.ds(s * STRIPE, STRIPE)],
                    den_hbm.at[c, pl.ds(s * STRIPE, STRIPE), :])


_sc_denom = pl.kernel(
    _sc_denom_body,
    out_type=jax.ShapeDtypeStruct((2, NPAD, HID), jnp.float32),
    mesh=_MESH,
    scratch_types=[
        pltpu.VMEM_SHARED((NPAD, HID), jnp.float32),
        pltpu.VMEM((W, HID), jnp.float32),
        pltpu.VMEM((W, HID), jnp.float32),
        pltpu.VMEM((W,), jnp.int32),
        pltpu.VMEM((W,), jnp.int32),
        pltpu.SemaphoreType.DMA,
        pltpu.SemaphoreType.DMA,
    ],
)


def _sc_scatter_body(msg_hbm, di_hbm, op_hbm, out_sp,
                     zbuf, mb0, mb1, ix0, ix1, ssem0, ssem1):
    c = lax.axis_index("c")
    s = lax.axis_index("s")
    wid = s * 2 + c
    z = jnp.zeros((16,), jnp.float32)

    def zrow(i, carry):
        for j in range(HID // 16):
            zbuf[i, pl.ds(j * 16, 16)] = z
        return carry

    lax.fori_loop(0, ZW, zrow, 0)

    for h in range(HEADS):
        for k in range(STRIPE // ZW):
            pltpu.sync_copy(zbuf, out_sp.at[pl.ds(s * STRIPE + k * ZW, ZW)])
        plsc.subcore_barrier()

        def win(w2, carry):
            for j, (mb, ixb, ssem) in ((0, (mb0, ix0, ssem0)), (1, (mb1, ix1, ssem1))):
                base = pl.multiple_of(wid * EPW + (w2 * 2 + j) * W, 8)

                @pl.when(w2 > 0)
                def _():
                    pltpu.make_async_copy(mb, out_sp.at[ixb], ssem).wait()

                pltpu.sync_copy(msg_hbm.at[pl.ds(base, W), pl.ds(h * HID, HID)], mb)
                pltpu.sync_copy(di_hbm.at[pl.ds(base, W)], ixb)
                pltpu.async_copy(mb, out_sp.at[ixb], ssem, add=True)
            return carry

        lax.fori_loop(0, NWIN // 2, win, 0)
        pltpu.make_async_copy(mb0, out_sp.at[ix0], ssem0).wait()
        pltpu.make_async_copy(mb1, out_sp.at[ix1], ssem1).wait()
        plsc.subcore_barrier()
        pltpu.sync_copy(out_sp.at[pl.ds(s * STRIPE, STRIPE)],
                        op_hbm.at[c * HEADS + h, pl.ds(s * STRIPE, STRIPE), :])


_sc_scatter = pl.kernel(
    _sc_scatter_body,
    out_type=jax.ShapeDtypeStruct((2 * HEADS, NPAD, HID), jnp.float32),
    mesh=_MESH,
    scratch_types=[
        pltpu.VMEM_SHARED((NPAD, HID), jnp.float32),
        pltpu.VMEM((ZW, HID), jnp.float32),
        pltpu.VMEM((W, HID), jnp.float32),
        pltpu.VMEM((W, HID), jnp.float32),
        pltpu.VMEM((W,), jnp.int32),
        pltpu.VMEM((W,), jnp.int32),
        pltpu.SemaphoreType.DMA,
        pltpu.SemaphoreType.DMA,
    ],
)


def _final_body(de, op, den, bo, out_o):
    pieces = [de[...]]
    for h in range(HEADS):
        num = op[h] + op[HEADS + h]
        dh = den[0, :, h:h + 1] + den[1, :, h:h + 1] + 1e-16
        v = num / dh + bo[:, h * HID:(h + 1) * HID]
        pieces.append(jax.nn.relu(v))
    out_o[...] = jnp.concatenate(pieces, axis=1)


def _final(de, op, den, bias_out):
    return pl.pallas_call(
        _final_body,
        grid=(N_DST // RB,),
        in_specs=[
            pl.BlockSpec((RB, HID), lambda i: (i, 0)),
            pl.BlockSpec((2 * HEADS, RB, HID), lambda i: (0, i, 0)),
            pl.BlockSpec((2, RB, HID), lambda i: (0, i, 0)),
            pl.BlockSpec((1, HH), lambda i: (0, 0)),
        ],
        out_specs=pl.BlockSpec((RB, HID + HH), lambda i: (i, 0)),
        out_shape=jax.ShapeDtypeStruct((N_DST, HID + HH), jnp.float32),
    )(de, op, den, bias_out.reshape(1, HH))


def kernel(src_x, dst_x, edge_index, W_src, b_src, W_dst, b_dst,
           W_l, b_l, W_r, b_r, att, bias_out):
    ei = jnp.pad(edge_index, ((0, 0), (0, EPAD - E)))
    si = ei[0]
    di = ei[1]
    de, xl, xr = _dense(src_x, dst_x, W_src, b_src, W_dst, b_dst, W_l, b_l, W_r, b_r)
    xls, xrs = _sc_gather(xl, xr, si, di)
    exa, msg = _alpha(xls, xrs, att)
    den = _sc_denom(exa, di)
    op = _sc_scatter(msg, di)
    return _final(de, op, den, bias_out)


# speedup vs baseline: 1.2987x; 1.0326x over previous
import jax
import jax.numpy as jnp
from jax import lax
from jax.experimental import pallas as pl
from jax.experimental.pallas import tpu as pltpu
from jax.experimental.pallas import tpu_sc as plsc

N_SRC = 10000
N_DST = 10000
E = 160000
SRC_DIMS = 256
DST_DIMS = 256
HID = 128
HEADS = 4
HH = HEADS * HID

NW = 32
W = 120
NWIN = 44
EPW = W * NWIN
EPAD = NW * EPW
NTILES = 16
NPAD = 10240
STRIPE = NPAD // NTILES
ZW = 80
EB = 960
RB = 1000

_MESH = plsc.VectorSubcoreMesh(core_axis_name="c", subcore_axis_name="s",
                               num_cores=2, num_subcores=16)


def _pack(x):
    lo = lax.bitcast_convert_type(x[:, :HH // 2], jnp.int32)
    hi = lax.bitcast_convert_type(x[:, HH // 2:], jnp.int32)
    plo = (lo + 0x8000) & jnp.int32(-65536)
    phi = lax.shift_right_logical(hi + 0x8000, 16)
    return plo | phi


def _unpack(p):
    lo = lax.bitcast_convert_type(p & jnp.int32(-65536), jnp.float32)
    hi = lax.bitcast_convert_type(lax.shift_left(p, 16), jnp.float32)
    return jnp.concatenate([lo, hi], axis=1)


def _dense_body(sx, dx, ws, bs, wd, bd, wl, bl, wr, br, de_o, xl_o, xr_o):
    se = jax.nn.relu(jnp.dot(sx[...], ws[...], preferred_element_type=jnp.float32) + bs[...])
    de = jax.nn.relu(jnp.dot(dx[...], wd[...], preferred_element_type=jnp.float32) + bd[...])
    de_o[...] = de
    xl = jnp.dot(se, wl[...], preferred_element_type=jnp.float32) + bl[...]
    xr = jnp.dot(de, wr[...], preferred_element_type=jnp.float32) + br[...]
    xl_o[...] = _pack(xl)
    xr_o[...] = _pack(xr)


def _dense(src_x, dst_x, W_src, b_src, W_dst, b_dst, W_l, b_l, W_r, b_r):
    full = lambda shape: pl.BlockSpec(shape, lambda i: (0,) * len(shape))
    return pl.pallas_call(
        _dense_body,
        grid=(N_SRC // RB,),
        in_specs=[
            pl.BlockSpec((RB, SRC_DIMS), lambda i: (i, 0)),
            pl.BlockSpec((RB, DST_DIMS), lambda i: (i, 0)),
            full((SRC_DIMS, HID)), full((1, HID)),
            full((DST_DIMS, HID)), full((1, HID)),
            full((HID, HH)), full((1, HH)),
            full((HID, HH)), full((1, HH)),
        ],
        out_specs=[
            pl.BlockSpec((RB, HID), lambda i: (i, 0)),
            pl.BlockSpec((RB, HH // 2), lambda i: (i, 0)),
            pl.BlockSpec((RB, HH // 2), lambda i: (i, 0)),
        ],
        out_shape=[
            jax.ShapeDtypeStruct((N_DST, HID), jnp.float32),
            jax.ShapeDtypeStruct((N_SRC, HH // 2), jnp.int32),
            jax.ShapeDtypeStruct((N_DST, HH // 2), jnp.int32),
        ],
    )(src_x, dst_x, W_src, b_src.reshape(1, HID), W_dst, b_dst.reshape(1, HID),
      W_l, b_l.reshape(1, HH), W_r, b_r.reshape(1, HH))


def _sc_gather_body(xl_hbm, xr_hbm, si_hbm, di_hbm, xls_hbm, xrs_hbm,
                    idx0, idx1, buf0, buf1, gsem0, gsem1, wsem0, wsem1):
    c = lax.axis_index("c")
    s = lax.axis_index("s")
    wid = s * 2 + c
    ebase = pl.multiple_of(wid * EPW, 8)
    pltpu.sync_copy(si_hbm.at[pl.ds(ebase, EPW)], idx0)
    pltpu.sync_copy(di_hbm.at[pl.ds(ebase, EPW)], idx1)

    def win(w, carry):
        base = pl.multiple_of(ebase + w * W, 8)
        dst0 = xls_hbm.at[pl.ds(base, W), :]
        dst1 = xrs_hbm.at[pl.ds(base, W), :]

        @pl.when(w > 0)
        def _():
            pltpu.make_async_copy(buf0, dst0, wsem0).wait()
            pltpu.make_async_copy(buf1, dst1, wsem1).wait()

        g0 = pltpu.async_copy(xl_hbm.at[idx0.at[pl.ds(w * W, W)]], buf0, gsem0)
        g1 = pltpu.async_copy(xr_hbm.at[idx1.at[pl.ds(w * W, W)]], buf1, gsem1)
        g0.wait()
        pltpu.async_copy(buf0, dst0, wsem0)
        g1.wait()
        pltpu.async_copy(buf1, dst1, wsem1)
        return carry

    lax.fori_loop(0, NWIN, win, 0)
    last = pl.multiple_of(wid * EPW + (NWIN - 1) * W, 8)
    pltpu.make_async_copy(buf0, xls_hbm.at[pl.ds(last, W), :], wsem0).wait()
    pltpu.make_async_copy(buf1, xrs_hbm.at[pl.ds(last, W), :], wsem1).wait()


_sc_gather = pl.kernel(
    _sc_gather_body,
    out_type=[
        jax.ShapeDtypeStruct((EPAD, HH // 2), jnp.int32),
        jax.ShapeDtypeStruct((EPAD, HH // 2), jnp.int32),
    ],
    mesh=_MESH,
    scratch_types=[
        pltpu.VMEM((EPW,), jnp.int32),
        pltpu.VMEM((EPW,), jnp.int32),
        pltpu.VMEM((W, HH // 2), jnp.int32),
        pltpu.VMEM((W, HH // 2), jnp.int32),
        pltpu.SemaphoreType.DMA,
        pltpu.SemaphoreType.DMA,
        pltpu.SemaphoreType.DMA,
        pltpu.SemaphoreType.DMA,
    ],
)


def _alpha_body(xls, xrs, att, ex_o, msg_o):
    pid = pl.program_id(0)
    xlv = _unpack(xls[...])
    xrv = _unpack(xrs[...])
    eid = pid * EB + lax.broadcasted_iota(jnp.int32, (EB, 1), 0)
    live = eid < E
    cols = []
    pieces = []
    for h in range(HEADS):
        xlh = xlv[:, h * HID:(h + 1) * HID]
        x = xlh + xrv[:, h * HID:(h + 1) * HID]
        x = jnp.where(x > 0, x, 0.2 * x)
        a = jnp.sum(x * att[h, :][None, :], axis=1, keepdims=True)
        ex = jnp.where(live, jnp.exp(a), 0.0)
        cols.append(ex)
        pieces.append(xlh * ex)
    ex_o[...] = jnp.concatenate(cols + [jnp.zeros((EB, HID - HEADS), jnp.float32)], axis=1)
    msg_o[...] = jnp.concatenate(pieces, axis=1)


def _alpha(xls, xrs, att):
    return pl.pallas_call(
        _alpha_body,
        grid=(EPAD // EB,),
        in_specs=[
            pl.BlockSpec((EB, HH // 2), lambda i: (i, 0)),
            pl.BlockSpec((EB, HH // 2), lambda i: (i, 0)),
            pl.BlockSpec((HEADS, HID), lambda i: (0, 0)),
        ],
        out_specs=[
            pl.BlockSpec((EB, HID), lambda i: (i, 0)),
            pl.BlockSpec((EB, HH), lambda i: (i, 0)),
        ],
        out_shape=[
            jax.ShapeDtypeStruct((EPAD, HID), jnp.float32),
            jax.ShapeDtypeStruct((EPAD, HH), jnp.float32),
        ],
    )(xls, xrs, att)


def _sc_denom_body(exa_hbm, di_hbm, den_hbm, den_sp,
                   ex0, ex1, ix0, ix1, ssem0, ssem1):
    c = lax.axis_index("c")
    s = lax.axis_index("s")
    wid2 = c * NTILES + s
    z = jnp.zeros((16,), jnp.float32)

    def zrow(i, carry):
        for j in range(HID // 16):
            ex0[i, pl.ds(j * 16, 16)] = z
        return carry

    lax.fori_loop(0, ZW, zrow, 0)
    for k in range(STRIPE // ZW):
        pltpu.sync_copy(ex0.at[pl.ds(0, ZW)], den_sp.at[pl.ds(s * STRIPE + k * ZW, ZW)])
    plsc.subcore_barrier()

    def win(w2, carry):
        for j, (exb, ixb, ssem) in ((0, (ex0, ix0, ssem0)), (1, (ex1, ix1, ssem1))):
            base = pl.multiple_of(wid2 * EPW + (w2 * 2 + j) * W, 8)

            @pl.when(w2 > 0)
            def _():
                pltpu.make_async_copy(exb, den_sp.at[ixb], ssem).wait()

            pltpu.sync_copy(exa_hbm.at[pl.ds(base, W), :], exb)
            pltpu.sync_copy(di_hbm.at[pl.ds(base, W)], ixb)
            pltpu.async_copy(exb, den_sp.at[ixb], ssem, add=True)
        return carry

    lax.fori_loop(0, NWIN // 2, win, 0)
    pltpu.make_async_copy(ex0, den_sp.at[ix0], ssem0).wait()
    pltpu.make_async_copy(ex1, den_sp.at[ix1], ssem1).wait()
    plsc.subcore_barrier()
    pltpu.sync_copy(den_sp.at[pl.ds(s * STRIPE, STRIPE)],
                    den_hbm.at[c, pl.ds(s * STRIPE, STRIPE), :])


_sc_denom = pl.kernel(
    _sc_denom_body,
    out_type=jax.ShapeDtypeStruct((2, NPAD, HID), jnp.float32),
    mesh=_MESH,
    scratch_types=[
        pltpu.VMEM_SHARED((NPAD, HID), jnp.float32),
        pltpu.VMEM((W, HID), jnp.float32),
        pltpu.VMEM((W, HID), jnp.float32),
        pltpu.VMEM((W,), jnp.int32),
        pltpu.VMEM((W,), jnp.int32),
        pltpu.SemaphoreType.DMA,
        pltpu.SemaphoreType.DMA,
    ],
)


def _sc_scatter_body(msg_hbm, di_hbm, op_hbm, out_sp,
                     zbuf, mb0, mb1, ix0, ix1, ssem0, ssem1):
    c = lax.axis_index("c")
    s = lax.axis_index("s")
    wid = s * 2 + c
    z = jnp.zeros((16,), jnp.float32)

    def zrow(i, carry):
        for j in range(HID // 16):
            zbuf[i, pl.ds(j * 16, 16)] = z
        return carry

    lax.fori_loop(0, ZW, zrow, 0)

    for h in range(HEADS):
        for k in range(STRIPE // ZW):
            pltpu.sync_copy(zbuf, out_sp.at[pl.ds(s * STRIPE + k * ZW, ZW)])
        plsc.subcore_barrier()

        def win(w2, carry):
            for j, (mb, ixb, ssem) in ((0, (mb0, ix0, ssem0)), (1, (mb1, ix1, ssem1))):
                base = pl.multiple_of(wid * EPW + (w2 * 2 + j) * W, 8)

                @pl.when(w2 > 0)
                def _():
                    pltpu.make_async_copy(mb, out_sp.at[ixb], ssem).wait()

                pltpu.sync_copy(msg_hbm.at[pl.ds(base, W), pl.ds(h * HID, HID)], mb)
                pltpu.sync_copy(di_hbm.at[pl.ds(base, W)], ixb)
                pltpu.async_copy(mb, out_sp.at[ixb], ssem, add=True)
            return carry

        lax.fori_loop(0, NWIN // 2, win, 0)
        pltpu.make_async_copy(mb0, out_sp.at[ix0], ssem0).wait()
        pltpu.make_async_copy(mb1, out_sp.at[ix1], ssem1).wait()
        plsc.subcore_barrier()
        pltpu.sync_copy(out_sp.at[pl.ds(s * STRIPE, STRIPE)],
                        op_hbm.at[c * HEADS + h, pl.ds(s * STRIPE, STRIPE), :])


_sc_scatter = pl.kernel(
    _sc_scatter_body,
    out_type=jax.ShapeDtypeStruct((2 * HEADS, NPAD, HID), jnp.float32),
    mesh=_MESH,
    scratch_types=[
        pltpu.VMEM_SHARED((NPAD, HID), jnp.float32),
        pltpu.VMEM((ZW, HID), jnp.float32),
        pltpu.VMEM((W, HID), jnp.float32),
        pltpu.VMEM((W, HID), jnp.float32),
        pltpu.VMEM((W,), jnp.int32),
        pltpu.VMEM((W,), jnp.int32),
        pltpu.SemaphoreType.DMA,
        pltpu.SemaphoreType.DMA,
    ],
)


def _final_body(de, op, den, bo, out_o):
    pieces = [de[...]]
    for h in range(HEADS):
        num = op[h] + op[HEADS + h]
        dh = den[0, :, h:h + 1] + den[1, :, h:h + 1] + 1e-16
        v = num / dh + bo[:, h * HID:(h + 1) * HID]
        pieces.append(jax.nn.relu(v))
    out_o[...] = jnp.concatenate(pieces, axis=1)


def _final(de, op, den, bias_out):
    return pl.pallas_call(
        _final_body,
        grid=(N_DST // RB,),
        in_specs=[
            pl.BlockSpec((RB, HID), lambda i: (i, 0)),
            pl.BlockSpec((2 * HEADS, RB, HID), lambda i: (0, i, 0)),
            pl.BlockSpec((2, RB, HID), lambda i: (0, i, 0)),
            pl.BlockSpec((1, HH), lambda i: (0, 0)),
        ],
        out_specs=pl.BlockSpec((RB, HID + HH), lambda i: (i, 0)),
        out_shape=jax.ShapeDtypeStruct((N_DST, HID + HH), jnp.float32),
    )(de, op, den, bias_out.reshape(1, HH))


def kernel(src_x, dst_x, edge_index, W_src, b_src, W_dst, b_dst,
           W_l, b_l, W_r, b_r, att, bias_out):
    ei = jnp.pad(edge_index, ((0, 0), (0, EPAD - E)))
    si = ei[0]
    di = ei[1]
    de, xl, xr = _dense(src_x, dst_x, W_src, b_src, W_dst, b_dst, W_l, b_l, W_r, b_r)
    xls, xrs = _sc_gather(xl, xr, si, di)
    exa, msg = _alpha(xls, xrs, att)
    den = _sc_denom(exa, di)
    op = _sc_scatter(msg, di)
    return _final(de, op, den, bias_out)
